# Initial kernel scaffold; baseline (speedup 1.0000x reference)
import jax
import jax.numpy as jnp
from jax.experimental import pallas as pl


def kernel(image, features, w_conv, b_conv, w_cls, b_cls, w_reg, b_reg, w_score, b_score, w_box, b_box):
    raise NotImplementedError("write your pallas kernel here")


# speedup vs baseline: 9.1821x; 9.1821x over previous
import functools

import jax
import jax.numpy as jnp
import numpy as np
from jax.experimental import pallas as pl
from jax.experimental.pallas import tpu as pltpu

IMG = 800
FH, FW = 50, 50
C = 256
A = 9
STRIDE = 16
ROIS = 300
PRE_NMS = 6000
NMS_T = 0.7
POOL = 7
NUM_CLASSES = 81
NEG = -1e30

N_ANC = FH * FW * A
N_PAD = 22528
ROWS = N_PAD // 128


def _np_anchors():
    sizes = np.array([128.0, 256.0, 512.0])
    ratios = np.array([0.5, 1.0, 2.0])
    ws = (sizes[None, :] * np.sqrt(1.0 / ratios)[:, None]).reshape(-1)
    hs = (sizes[None, :] * np.sqrt(ratios)[:, None]).reshape(-1)
    cy = (np.arange(FH) + 0.5) * STRIDE
    cx = (np.arange(FW) + 0.5) * STRIDE
    cyg, cxg = np.meshgrid(cy, cx, indexing='ij')
    cyg = cyg[:, :, None]
    cxg = cxg[:, :, None]
    anc = np.stack([cxg - ws / 2, cyg - hs / 2, cxg + ws / 2, cyg + hs / 2],
                   axis=-1).reshape(-1, 4)
    return anc.astype(np.float32)

_ANC = _np_anchors()
_AW = _ANC[:, 2] - _ANC[:, 0]
_AH = _ANC[:, 3] - _ANC[:, 1]
_ACX = _ANC[:, 0] + np.float32(0.5) * _AW
_ACY = _ANC[:, 1] + np.float32(0.5) * _AH


def _pad_plane(v, fill):
    out = np.full((N_PAD,), fill, np.float32)
    out[:N_ANC] = v
    return out.reshape(ROWS, 128)

_ANC_PLANES = np.stack([
    _pad_plane(_ACX, 0.0),
    _pad_plane(_ACY, 0.0),
    _pad_plane(_AW, 1.0),
    _pad_plane(_AH, 1.0),
])


def _conv_kernel(x_ref, w_ref, wcr_ref, bc_ref, bcr_ref, acc_ref, z_ref):
    k = pl.program_id(0)

    @pl.when(k == 0)
    def _init():
        acc_ref[...] = jnp.zeros_like(acc_ref)

    acc_ref[...] += jax.lax.dot(
        x_ref[0], w_ref[0], preferred_element_type=jnp.float32)

    @pl.when(k == 8)
    def _heads():
        rpn = jnp.maximum(acc_ref[...] + bc_ref[...], 0.0)
        z_ref[...] = jax.lax.dot(
            rpn, wcr_ref[...], preferred_element_type=jnp.float32) + bcr_ref[...]


def _conv_stage(features, w_conv, b_conv, w_cls, b_cls, w_reg, b_reg):
    xpad = jnp.pad(features[0], ((1, 1), (1, 1), (0, 0)))
    xs = jnp.stack([
        xpad[dy:dy + FH, dx:dx + FW].reshape(FH * FW, C)
        for dy in range(3) for dx in range(3)
    ])
    wk = w_conv.reshape(9, C, 512)
    wcr = jnp.concatenate([w_cls[0, 0], w_reg[0, 0]], axis=1)
    wcr = jnp.pad(wcr, ((0, 0), (0, 10)))
    bcr = jnp.pad(jnp.concatenate([b_cls, b_reg]), (0, 10)).reshape(1, 64)

    z = pl.pallas_call(
        _conv_kernel,
        grid=(9,),
        in_specs=[
            pl.BlockSpec((1, FH * FW, C), lambda k: (k, 0, 0)),
            pl.BlockSpec((1, C, 512), lambda k: (k, 0, 0)),
            pl.BlockSpec((512, 64), lambda k: (0, 0)),
            pl.BlockSpec((1, 512), lambda k: (0, 0)),
            pl.BlockSpec((1, 64), lambda k: (0, 0)),
        ],
        out_specs=[
            pl.BlockSpec((FH * FW, 512), lambda k: (0, 0)),
            pl.BlockSpec((FH * FW, 64), lambda k: (0, 0)),
        ],
        out_shape=[
            jax.ShapeDtypeStruct((FH * FW, 512), jnp.float32),
            jax.ShapeDtypeStruct((FH * FW, 64), jnp.float32),
        ],
    )(xs, wk, wcr, b_conv.reshape(1, 512), bcr)[1]
    return z


def _nms_kernel(obj_ref, box_ref, out_ref, s_ref, area_ref):
    obj = obj_ref[...]
    si = jax.lax.bitcast_convert_type(obj, jnp.int32)
    idx2 = jax.lax.broadcasted_iota(jnp.int32, (ROWS, 128), 0) * 128 \
        + jax.lax.broadcasted_iota(jnp.int32, (ROWS, 128), 1)

    def _count(pred):
        return jnp.sum(pred.astype(jnp.int32))

    one_bits = jnp.int32(0x3F800001)

    def _bs_val(_, lohi):
        lo, hi = lohi
        mid = lo + (hi - lo) // 2
        ge = _count(si >= mid) >= PRE_NMS
        return jnp.where(ge, mid, lo), jnp.where(ge, hi, mid)

    thr, _ = jax.lax.fori_loop(
        0, 31, _bs_val, (jnp.int32(0), one_bits))
    n_gt = _count(si >= thr + 1)
    needed = PRE_NMS - n_gt
    eq = si == thr

    def _bs_idx(_, lohi):
        lo, hi = lohi
        mid = lo + (hi - lo) // 2
        ge = _count(eq & (idx2 < mid)) >= needed
        return jnp.where(ge, lo, mid), jnp.where(ge, mid, hi)

    _, m_cut = jax.lax.fori_loop(
        0, 16, _bs_idx, (jnp.int32(0), jnp.int32(N_PAD)))

    keep_mask = (si > thr) | (eq & (idx2 < m_cut))
    neg = jnp.float32(NEG)
    s_ref[...] = jnp.where(keep_mask, obj, neg)
    area_ref[...] = (box_ref[2] - box_ref[0]) * (box_ref[3] - box_ref[1])

    top0 = jnp.min(jnp.where(obj == jnp.max(obj), idx2, N_PAD))

    lane = jax.lax.broadcasted_iota(jnp.int32, (1, 128), 1)

    def _step(t, top0):
        s = s_ref[...]
        mx = jnp.max(s)
        idx = jnp.min(jnp.where(s == mx, idx2, N_PAD))
        idx = jnp.where(mx > -1e29, idx, top0)
        r = idx // 128
        c = idx - r * 128
        lm = lane == c
        bx1 = jnp.sum(jnp.where(lm, box_ref[0, pl.ds(r, 1), :], 0.0))
        by1 = jnp.sum(jnp.where(lm, box_ref[1, pl.ds(r, 1), :], 0.0))
        bx2 = jnp.sum(jnp.where(lm, box_ref[2, pl.ds(r, 1), :], 0.0))
        by2 = jnp.sum(jnp.where(lm, box_ref[3, pl.ds(r, 1), :], 0.0))
        ix1 = jnp.maximum(bx1, box_ref[0])
        iy1 = jnp.maximum(by1, box_ref[1])
        ix2 = jnp.minimum(bx2, box_ref[2])
        iy2 = jnp.minimum(by2, box_ref[3])
        inter = jnp.maximum(ix2 - ix1, 0.0) * jnp.maximum(iy2 - iy1, 0.0)
        a1 = (bx2 - bx1) * (by2 - by1)
        iou = inter / (a1 + area_ref[...] - inter + 1e-8)
        s = jnp.where(iou >= NMS_T, neg, s)
        s_ref[...] = jnp.where(idx2 == idx, neg, s)
        row = jnp.where(lane == 0, bx1,
              jnp.where(lane == 1, by1,
              jnp.where(lane == 2, bx2,
              jnp.where(lane == 3, by2, 0.0))))
        out_ref[pl.ds(t, 1), :] = row
        return top0

    jax.lax.fori_loop(0, ROIS, _step, top0)
    out_ref[ROIS:, :] = jnp.zeros((304 - ROIS, 128), jnp.float32)


def _nms_stage(obj, props):
    obj_p = jnp.pad(obj, (0, N_PAD - N_ANC), constant_values=-1.0)
    box_p = jnp.pad(props.T, ((0, 0), (0, N_PAD - N_ANC)))
    sel = pl.pallas_call(
        _nms_kernel,
        out_shape=jax.ShapeDtypeStruct((304, 128), jnp.float32),
        scratch_shapes=[
            pltpu.VMEM((ROWS, 128), jnp.float32),
            pltpu.VMEM((ROWS, 128), jnp.float32),
        ],
    )(obj_p.reshape(ROWS, 128), box_p.reshape(4, ROWS, 128))
    return sel


RB = 304
FLATP = 160

_GRID_C = np.asarray((np.arange(POOL, dtype=np.float32)
                      + np.float32(0.5)) / np.float32(POOL))


def _roi_kernel(sel_ref, img_ref, ws_ref, bs_ref, wb_ref, bb_ref,
                score_ref, boxes_ref, wx_ref, flat_ref):
    x1 = sel_ref[:, 0:1]
    y1 = sel_ref[:, 1:2]
    x2 = sel_ref[:, 2:3]
    y2 = sel_ref[:, 3:4]
    cols = jax.lax.broadcasted_iota(jnp.int32, (RB, IMG), 1)
    lane128 = jax.lax.broadcasted_iota(jnp.int32, (1, 128), 1)

    def interp_mat(lo, hi, g):
        gpt = lo + g * (hi - lo)
        p0 = jnp.clip(jnp.floor(gpt), 0.0, IMG - 2.0)
        w = jnp.clip(gpt - p0, 0.0, 1.0)
        p0i = p0.astype(jnp.int32)
        return jnp.where(cols == p0i, 1.0 - w,
                         jnp.where(cols == p0i + 1, w, 0.0))

    for j in range(POOL):
        wx_ref[j] = interp_mat(x1, x2, _GRID_C[j])

    flat_ref[...] = jnp.zeros_like(flat_ref)
    for i in range(POOL):
        wy = interp_mat(y1, y2, _GRID_C[i])
        for c in range(3):
            t = jax.lax.dot(wy, img_ref[c],
                            preferred_element_type=jnp.float32)
            for j in range(POOL):
                k = (i * POOL + j) * 3 + c
                flat_ref[:, k:k + 1] = jnp.sum(
                    t * wx_ref[j], axis=1, keepdims=True)

    flat = flat_ref[...]
    logits = jax.lax.dot(flat, ws_ref[...],
                         preferred_element_type=jnp.float32) + bs_ref[...]
    logits = jnp.where(lane128 < NUM_CLASSES, logits, -jnp.inf)
    mx = jnp.max(logits, axis=1, keepdims=True)
    ex = jnp.exp(logits - mx)
    score_ref[...] = ex / jnp.sum(ex, axis=1, keepdims=True)
    boxes_ref[...] = jax.lax.dot(flat, wb_ref[...],
                                 preferred_element_type=jnp.float32) + bb_ref[...]


def _roi_stage(sel_raw, image, w_score, b_score, w_box, b_box):
    img3 = jnp.transpose(image[0], (2, 0, 1))
    ws = jnp.pad(w_score, ((0, FLATP - 147), (0, 128 - NUM_CLASSES)))
    bs = jnp.pad(b_score, (0, 128 - NUM_CLASSES)).reshape(1, 128)
    wb = jnp.pad(w_box, ((0, FLATP - 147), (0, 384 - 4 * NUM_CLASSES)))
    bb = jnp.pad(b_box, (0, 384 - 4 * NUM_CLASSES)).reshape(1, 384)
    score, boxes = pl.pallas_call(
        _roi_kernel,
        out_shape=[
            jax.ShapeDtypeStruct((RB, 128), jnp.float32),
            jax.ShapeDtypeStruct((RB, 384), jnp.float32),
        ],
        scratch_shapes=[
            pltpu.VMEM((POOL, RB, IMG), jnp.float32),
            pltpu.VMEM((RB, FLATP), jnp.float32),
        ],
    )(sel_raw, img3, ws, bs, wb, bb)
    return score[:ROIS, :NUM_CLASSES], boxes[:ROIS, :4 * NUM_CLASSES]


def _jx_iou(box, boxes):
    x1 = jnp.maximum(box[0], boxes[:, 0])
    y1 = jnp.maximum(box[1], boxes[:, 1])
    x2 = jnp.minimum(box[2], boxes[:, 2])
    y2 = jnp.minimum(box[3], boxes[:, 3])
    inter = jnp.clip(x2 - x1, 0.0) * jnp.clip(y2 - y1, 0.0)
    a1 = (box[2] - box[0]) * (box[3] - box[1])
    a2 = (boxes[:, 2] - boxes[:, 0]) * (boxes[:, 3] - boxes[:, 1])
    return inter / (a1 + a2 - inter + 1e-8)


def _jx_nms(boxes, scores):
    def step(s, _):
        idx = jnp.argmax(s)
        iou = _jx_iou(boxes[idx], boxes)
        s2 = jnp.where(iou >= NMS_T, NEG, s)
        s2 = s2.at[idx].set(NEG)
        return s2, idx
    _, keep = jax.lax.scan(step, scores, None, length=ROIS)
    return keep


def _decode(z):
    cls = jax.nn.sigmoid(z[:, :18])
    reg = z[:, 18:54]
    deltas = reg.reshape(-1, 4)
    obj = cls.reshape(FH, FW, A, 2)[..., 1].reshape(-1)
    aw = jnp.asarray(_AW)
    ah = jnp.asarray(_AH)
    acx = jnp.asarray(_ACX)
    acy = jnp.asarray(_ACY)
    cx = acx + aw * deltas[:, 0]
    cy = acy + ah * deltas[:, 1]
    w = aw * jnp.exp(jnp.clip(deltas[:, 2], -5.0, 5.0))
    h = ah * jnp.exp(jnp.clip(deltas[:, 3], -5.0, 5.0))
    props = jnp.stack([cx - 0.5 * w, cy - 0.5 * h, cx + 0.5 * w, cy + 0.5 * h],
                      axis=1)
    props = jnp.clip(props, 0.0, IMG - 1.0)
    return cls, reg, obj, props


def _jx_roi_tail(sel, image, w_score, b_score, w_box, b_box):
    img = image[0]
    x1, y1, x2, y2 = sel[:, 0], sel[:, 1], sel[:, 2], sel[:, 3]
    gy = y1[:, None] + (jnp.arange(POOL, dtype=jnp.float32) + 0.5) / POOL * (y2 - y1)[:, None]
    gx = x1[:, None] + (jnp.arange(POOL, dtype=jnp.float32) + 0.5) / POOL * (x2 - x1)[:, None]
    yy = gy[:, :, None]
    xx = gx[:, None, :]
    y0 = jnp.clip(jnp.floor(yy), 0.0, IMG - 2.0)
    x0 = jnp.clip(jnp.floor(xx), 0.0, IMG - 2.0)
    wy = jnp.clip(yy - y0, 0.0, 1.0)[..., None]
    wx = jnp.clip(xx - x0, 0.0, 1.0)[..., None]
    y0i = y0.astype(jnp.int32)
    x0i = x0.astype(jnp.int32)
    v00 = img[y0i, x0i]
    v01 = img[y0i, x0i + 1]
    v10 = img[y0i + 1, x0i]
    v11 = img[y0i + 1, x0i + 1]
    pooled = (v00 * (1 - wy) * (1 - wx) + v01 * (1 - wy) * wx
              + v10 * wy * (1 - wx) + v11 * wy * wx)
    flat = pooled.reshape(ROIS, POOL * POOL * 3)
    score = jax.nn.softmax(flat @ w_score + b_score, axis=-1)
    boxes_out = flat @ w_box + b_box
    return score, boxes_out


def kernel(image, features, w_conv, b_conv, w_cls, b_cls, w_reg, b_reg,
           w_score, b_score, w_box, b_box):
    z = _conv_stage(features, w_conv, b_conv, w_cls, b_cls, w_reg, b_reg)
    cls, reg, obj, props = _decode(z)
    sel_raw = _nms_stage(obj, props)
    score, boxes_out = _roi_stage(sel_raw, image, w_score, b_score, w_box, b_box)
    rpn_prediction = jnp.concatenate([cls, reg], axis=-1).reshape(1, FH, FW, 18 + 36)
    return rpn_prediction, score, boxes_out
